# baseline (speedup 1.0000x reference)
import functools

import jax
import jax.numpy as jnp
from jax import lax
from jax.experimental import pallas as pl
from jax.experimental.pallas import tpu as pltpu
from jax.experimental.pallas import tpu_sc as plsc

_W = 16
_WIN_PER_TILE = 128
_CHUNK = 64
_SLAB_ROWS = _CHUNK + _W
_NRING = 2
_NLANE = 16


def kernel(x):
    B, S, D = x.shape
    n_win = S - _W + 1
    n_chunks = _WIN_PER_TILE // _CHUNK

    info = plsc.get_sparse_core_info()
    nc, ns = info.num_cores, info.num_subcores
    n_workers = nc * ns
    lanes_per_batch = n_workers // B

    mesh = plsc.VectorSubcoreMesh(core_axis_name="c", subcore_axis_name="s")

    @functools.partial(
        pl.kernel,
        mesh=mesh,
        out_type=jax.ShapeDtypeStruct((B, n_win, _W, D), x.dtype),
        scratch_types=[
            pltpu.VMEM((_SLAB_ROWS, D), x.dtype),
            pltpu.VMEM((_NRING, _W, D), x.dtype),
            pltpu.SemaphoreType.DMA,
            pltpu.SemaphoreType.DMA,
        ],
    )
    def win_align(x_hbm, out_hbm, slab_v, ring_v, in_sem, out_sem):
        c = lax.axis_index("c")
        s = lax.axis_index("s")
        wid = s * nc + c
        b = wid // lanes_per_batch
        lane = wid % lanes_per_batch
        w0 = lane * _WIN_PER_TILE
        cnt = jnp.minimum(_WIN_PER_TILE, n_win - w0)

        def out_copy(r, w):
            return pltpu.make_async_copy(
                ring_v.at[r], out_hbm.at[b, w, :, :], out_sem
            )

        def do_chunk(chunk, carry):
            c0 = w0 + chunk * _CHUNK
            ccnt = jnp.minimum(_CHUNK, cnt - chunk * _CHUNK)
            s0 = pl.multiple_of(jnp.minimum(c0, S - _SLAB_ROWS), 8)
            off = c0 - s0
            mbase = lax.shift_right_logical(off, 3)
            pltpu.async_copy(
                x_hbm.at[b, pl.ds(s0, _SLAB_ROWS), :], slab_v, in_sem
            ).wait()

            def phase_body(p, carry):
                kcnt = lax.div(ccnt - p + 7, 8)

                def k_body(k, carry):
                    r = lax.rem(k, _NRING)

                    @pl.when(k >= _NRING)
                    def _():
                        out_copy(r, c0 + p + 8 * (k - _NRING)).wait()

                    base = 8 * (mbase + k) + p
                    for j in range(_W):
                        for l in range(D // _NLANE):
                            ring_v[r, j, pl.ds(l * _NLANE, _NLANE)] = (
                                slab_v[
                                    base + j,
                                    pl.ds(l * _NLANE, _NLANE),
                                ]
                            )
                    out_copy(r, c0 + p + 8 * k).start()
                    return carry

                lax.fori_loop(0, kcnt, k_body, 0)

                def drain(k, carry):
                    out_copy(lax.rem(k, _NRING), c0 + p + 8 * k).wait()
                    return carry

                lax.fori_loop(jnp.maximum(kcnt - _NRING, 0), kcnt, drain, 0)
                return carry

            lax.fori_loop(0, 8, phase_body, 0)
            return carry

        lax.fori_loop(0, n_chunks, do_chunk, 0)

    return win_align(x)


# speedup vs baseline: 1.5358x; 1.5358x over previous
import functools

import jax
import jax.numpy as jnp
from jax import lax
from jax.experimental import pallas as pl
from jax.experimental.pallas import tpu as pltpu
from jax.experimental.pallas import tpu_sc as plsc

_W = 16
_WIN_PER_TILE = 128
_LAG = 32


def kernel(x):
    B, S, D = x.shape
    n_win = S - _W + 1
    rows_per_tile = _WIN_PER_TILE + _W - 1

    info = plsc.get_sparse_core_info()
    nc, ns = info.num_cores, info.num_subcores
    n_workers = nc * ns
    lanes_per_batch = n_workers // B

    mesh = plsc.VectorSubcoreMesh(core_axis_name="c", subcore_axis_name="s")

    @functools.partial(
        pl.kernel,
        mesh=mesh,
        out_type=jax.ShapeDtypeStruct((B, n_win, _W, D), x.dtype),
        scratch_types=[
            pltpu.VMEM((rows_per_tile, D), x.dtype),
            pltpu.SemaphoreType.DMA,
            pltpu.SemaphoreType.DMA,
        ],
        compiler_params=pltpu.CompilerParams(use_tc_tiling_on_sc=False),
    )
    def win_align(x_hbm, out_hbm, rows_v, in_sem, out_sem):
        c = lax.axis_index("c")
        s = lax.axis_index("s")
        wid = s * nc + c
        b = wid // lanes_per_batch
        lane = wid % lanes_per_batch
        w0 = jnp.minimum(lane * _WIN_PER_TILE, n_win - _WIN_PER_TILE)

        pltpu.async_copy(
            x_hbm.at[b, pl.ds(w0, rows_per_tile), :], rows_v, in_sem
        ).wait()

        def window_copy(i):
            return pltpu.make_async_copy(
                rows_v.at[pl.ds(i, _W), :],
                out_hbm.at[b, w0 + i, :, :],
                out_sem,
            )

        def body(i, carry):
            window_copy(i).start()

            @pl.when(i >= _LAG)
            def _():
                window_copy(i - _LAG).wait()

            return carry

        lax.fori_loop(0, _WIN_PER_TILE, body, 0)

        def tail(i, carry):
            window_copy(i).wait()
            return carry

        lax.fori_loop(_WIN_PER_TILE - _LAG, _WIN_PER_TILE, tail, 0)

    return win_align(x)
